# baseline (speedup 1.0000x reference)
import functools

import jax
import jax.numpy as jnp
from jax import lax
from jax.experimental import pallas as pl
from jax.experimental.pallas import tpu as pltpu
from jax.experimental.pallas import tpu_sc as plsc

B, N, E, H = 4, 2048, 65536, 64
BN = B * N
BE = B * E
NC, NS, L = 2, 16, 16
ROWS_PER_CORE = BN // NC
CH = 512
CHUNKS = BE // (NC * NS * CH)
IDX_W = 128

_mesh = plsc.VectorSubcoreMesh(core_axis_name="c", subcore_axis_name="s")


def _silu(x):
    return x / (1.0 + jnp.exp(-x))


def _silu_vec(x):
    return x * (1.0 / (1.0 + jnp.exp(-x)))


def _proj2_body(h_ref, wp_ref, bp_ref, wq_ref, p_ref, q_ref):
    x = h_ref[...]
    p_ref[...] = lax.dot_general(x, wp_ref[...], (((1,), (0,)), ((), ())),
                                 preferred_element_type=jnp.float32) + bp_ref[...]
    q_ref[...] = lax.dot_general(x, wq_ref[...], (((1,), (0,)), ((), ())),
                                 preferred_element_type=jnp.float32)


def _proj2(h, wp, bp, wq):
    return pl.pallas_call(
        _proj2_body,
        out_shape=(jax.ShapeDtypeStruct((BN, H), jnp.float32),
                   jax.ShapeDtypeStruct((BN, H), jnp.float32)),
    )(h, wp, bp, wq)


def _update_body(h_ref, a_ref, w_ref, b_ref, o_ref):
    x = jnp.concatenate([h_ref[...], a_ref[...]], axis=1)
    o = lax.dot_general(x, w_ref[...], (((1,), (0,)), ((), ())),
                        preferred_element_type=jnp.float32)
    o_ref[...] = _silu(o + b_ref[...])


def _update(h, agg, w, b):
    return pl.pallas_call(
        _update_body,
        out_shape=jax.ShapeDtypeStruct((BN, H), jnp.float32),
    )(h, agg, w, b)


def _edge_agg_body(p_hbm, q_hbm, srcb_hbm, dstb_hbm, out_hbm,
                   idxs, idxd, idxw, rows_p, rows_q, agg_sh, semp, semq):
    c = lax.axis_index("c")
    s = lax.axis_index("s")
    zero16 = jnp.zeros((L,), jnp.float32)
    my_sh0 = s * (ROWS_PER_CORE // NS)

    @pl.loop(0, ROWS_PER_CORE // NS)
    def _z(r):
        for k in range(H // L):
            rows_p[r, pl.ds(k * L, L)] = zero16
    pltpu.sync_copy(rows_p.at[pl.ds(0, ROWS_PER_CORE // NS)],
                    agg_sh.at[pl.ds(my_sh0, ROWS_PER_CORE // NS)])
    plsc.subcore_barrier()

    base_row = (c * NS + s) * (CHUNKS * CH // IDX_W)
    row_off = c * ROWS_PER_CORE

    @pl.loop(0, CHUNKS)
    def _chunk(g):
        row0 = base_row + g * (CH // IDX_W)
        pltpu.sync_copy(srcb_hbm.at[pl.ds(row0, CH // IDX_W)], idxs)
        pltpu.sync_copy(dstb_hbm.at[pl.ds(row0, CH // IDX_W)], idxd)
        for j in range(CH // IDX_W):
            for k in range(IDX_W // L):
                sl = pl.ds(k * L, L)
                idxw[j, sl] = idxd[j, sl] - row_off
        descs = []
        for j in range(CH // IDX_W):
            descs.append(pltpu.async_copy(
                p_hbm.at[idxs.at[j]], rows_p.at[pl.ds(j * IDX_W, IDX_W)], semp))
            descs.append(pltpu.async_copy(
                q_hbm.at[idxd.at[j]], rows_q.at[pl.ds(j * IDX_W, IDX_W)], semq))
        for d in descs:
            d.wait()

        @pl.loop(0, CH)
        def _compute(r):
            for k in range(H // L):
                sl = pl.ds(k * L, L)
                a = rows_p[r, sl] + rows_q[r, sl]
                rows_p[r, sl] = _silu_vec(a)

        for j in range(CH // IDX_W):
            pltpu.sync_copy(rows_p.at[pl.ds(j * IDX_W, IDX_W)],
                            agg_sh.at[idxw.at[j]], add=True)

    plsc.subcore_barrier()
    pltpu.sync_copy(agg_sh.at[pl.ds(my_sh0, ROWS_PER_CORE // NS)],
                    out_hbm.at[pl.ds(row_off + my_sh0, ROWS_PER_CORE // NS)])


_edge_agg = pl.kernel(
    _edge_agg_body,
    out_type=jax.ShapeDtypeStruct((BN, H), jnp.float32),
    mesh=_mesh,
    compiler_params=pltpu.CompilerParams(use_tc_tiling_on_sc=False),
    scratch_types=[
        pltpu.VMEM((CH // IDX_W, IDX_W), jnp.int32),
        pltpu.VMEM((CH // IDX_W, IDX_W), jnp.int32),
        pltpu.VMEM((CH // IDX_W, IDX_W), jnp.int32),
        pltpu.VMEM((CH, H), jnp.float32),
        pltpu.VMEM((CH, H), jnp.float32),
        pltpu.VMEM_SHARED((ROWS_PER_CORE, H), jnp.float32),
        pltpu.SemaphoreType.DMA,
        pltpu.SemaphoreType.DMA,
    ],
)


def kernel(mu, t, context, edge_index, mp_params, edge_w1, edge_b1, edge_w2, edge_b2):
    t_exp = jnp.broadcast_to(t, (B, N))
    base = jnp.stack([mu, t_exp], axis=-1)
    h = jnp.concatenate([base, context], axis=-1).reshape(BN, -1)

    src = edge_index[0]
    dst = edge_index[1]
    offsets = (jnp.arange(B) * N).astype(src.dtype)
    src_b = (src[None, :] + offsets[:, None]).reshape(-1).astype(jnp.int32)
    dst_b = (dst[None, :] + offsets[:, None]).reshape(-1).astype(jnp.int32)
    srcb2 = src_b.reshape(BE // IDX_W, IDX_W)
    dstb2 = dst_b.reshape(BE // IDX_W, IDX_W)

    for (Wm, bm, Wn, bn) in mp_params:
        in_dim = h.shape[1]
        P, Q = _proj2(h, Wm[:in_dim], bm, Wm[in_dim:])
        agg = _edge_agg(P, Q, srcb2, dstb2)
        h = _update(h, agg, Wn, bn)

    U, V = _proj2(h, edge_w1[:H], edge_b1, edge_w1[H:])
    mid = _silu(U[src_b] + V[dst_b])
    logits = (mid @ edge_w2 + edge_b2).squeeze(-1)
    rates = jax.nn.softplus(logits).reshape(B, E)

    rm = jnp.zeros((B, N, N), jnp.float32)
    rm = rm.at[:, src, dst].set(rates)
    diag = jnp.arange(N)
    row_sum = rm.sum(axis=-1)
    rm = rm.at[:, diag, diag].set(-row_sum)
    return rm


# speedup vs baseline: 1.1478x; 1.1478x over previous
import functools

import jax
import jax.numpy as jnp
from jax import lax
from jax.experimental import pallas as pl
from jax.experimental.pallas import tpu as pltpu
from jax.experimental.pallas import tpu_sc as plsc

B, N, E, H = 4, 2048, 65536, 64
BN = B * N
BE = B * E
NC, NS, L = 2, 16, 16
ROWS_PER_CORE = BN // NC
CH = 512
CHUNKS = BE // (NC * NS * CH)
IDX_W = 128

_mesh = plsc.VectorSubcoreMesh(core_axis_name="c", subcore_axis_name="s")


def _silu(x):
    return x / (1.0 + jnp.exp(-x))


def _silu_vec(x):
    return x * (1.0 / (1.0 + jnp.exp(-x)))


def _proj2_body(h_ref, wp_ref, bp_ref, wq_ref, p_ref, q_ref):
    x = h_ref[...]
    p_ref[...] = lax.dot_general(x, wp_ref[...], (((1,), (0,)), ((), ())),
                                 preferred_element_type=jnp.float32) + bp_ref[...]
    q_ref[...] = lax.dot_general(x, wq_ref[...], (((1,), (0,)), ((), ())),
                                 preferred_element_type=jnp.float32)


def _proj2(h, wp, bp, wq):
    return pl.pallas_call(
        _proj2_body,
        out_shape=(jax.ShapeDtypeStruct((BN, H), jnp.float32),
                   jax.ShapeDtypeStruct((BN, H), jnp.float32)),
    )(h, wp, bp, wq)


def _update_body(h_ref, a_ref, w_ref, b_ref, o_ref):
    x = jnp.concatenate([h_ref[...], a_ref[...]], axis=1)
    o = lax.dot_general(x, w_ref[...], (((1,), (0,)), ((), ())),
                        preferred_element_type=jnp.float32)
    o_ref[...] = _silu(o + b_ref[...])


def _update(h, agg, w, b):
    return pl.pallas_call(
        _update_body,
        out_shape=jax.ShapeDtypeStruct((BN, H), jnp.float32),
    )(h, agg, w, b)


def _edge_agg_body(p_hbm, q_hbm, srcb_hbm, dstb_hbm, out_hbm,
                   idxs, idxd, idxw, rows_p, rows_q, agg_sh, semp, semq):
    c = lax.axis_index("c")
    s = lax.axis_index("s")
    zero16 = jnp.zeros((L,), jnp.float32)
    my_sh0 = s * (ROWS_PER_CORE // NS)

    @pl.loop(0, ROWS_PER_CORE // NS)
    def _z(r):
        for k in range(H // L):
            rows_p[r, pl.ds(k * L, L)] = zero16
    pltpu.sync_copy(rows_p.at[pl.ds(0, ROWS_PER_CORE // NS)],
                    agg_sh.at[pl.ds(my_sh0, ROWS_PER_CORE // NS)])
    plsc.subcore_barrier()

    base_row = (c * NS + s) * (CHUNKS * CH // IDX_W)
    row_off = c * ROWS_PER_CORE

    @pl.loop(0, CHUNKS)
    def _chunk(g):
        row0 = base_row + g * (CH // IDX_W)
        pltpu.sync_copy(srcb_hbm.at[pl.ds(row0, CH // IDX_W)], idxs)
        pltpu.sync_copy(dstb_hbm.at[pl.ds(row0, CH // IDX_W)], idxd)
        for j in range(CH // IDX_W):
            for k in range(IDX_W // L):
                sl = pl.ds(k * L, L)
                idxw[j, sl] = idxd[j, sl] - row_off
        descs = []
        for j in range(CH // IDX_W):
            descs.append(pltpu.async_copy(
                p_hbm.at[idxs.at[j]], rows_p.at[pl.ds(j * IDX_W, IDX_W)], semp))
            descs.append(pltpu.async_copy(
                q_hbm.at[idxd.at[j]], rows_q.at[pl.ds(j * IDX_W, IDX_W)], semq))
        for d in descs:
            d.wait()

        @pl.loop(0, CH)
        def _compute(r):
            for k in range(H // L):
                sl = pl.ds(k * L, L)
                a = rows_p[r, sl] + rows_q[r, sl]
                rows_p[r, sl] = _silu_vec(a)

        for j in range(CH // IDX_W):
            pltpu.sync_copy(rows_p.at[pl.ds(j * IDX_W, IDX_W)],
                            agg_sh.at[idxw.at[j]], add=True)

    plsc.subcore_barrier()
    pltpu.sync_copy(agg_sh.at[pl.ds(my_sh0, ROWS_PER_CORE // NS)],
                    out_hbm.at[pl.ds(row_off + my_sh0, ROWS_PER_CORE // NS)])


_edge_agg = pl.kernel(
    _edge_agg_body,
    out_type=jax.ShapeDtypeStruct((BN, H), jnp.float32),
    mesh=_mesh,
    compiler_params=pltpu.CompilerParams(use_tc_tiling_on_sc=False),
    scratch_types=[
        pltpu.VMEM((CH // IDX_W, IDX_W), jnp.int32),
        pltpu.VMEM((CH // IDX_W, IDX_W), jnp.int32),
        pltpu.VMEM((CH // IDX_W, IDX_W), jnp.int32),
        pltpu.VMEM((CH, H), jnp.float32),
        pltpu.VMEM((CH, H), jnp.float32),
        pltpu.VMEM_SHARED((ROWS_PER_CORE, H), jnp.float32),
        pltpu.SemaphoreType.DMA,
        pltpu.SemaphoreType.DMA,
    ],
)


def _edge_rate_body(u_hbm, v_hbm, srcb_hbm, dstb_hbm, w2_hbm, segidx_hbm,
                    out_hbm, idxs, idxd, rows_u, rows_v, zbuf, zrow, segidx,
                    w2v, acc_sh, semu, semv):
    c = lax.axis_index("c")
    s = lax.axis_index("s")
    pltpu.sync_copy(w2_hbm, w2v)
    pltpu.sync_copy(segidx_hbm, segidx)
    zero16 = jnp.zeros((L,), jnp.float32)
    my_acc0 = s * CH
    @pl.loop(0, CH * L // IDX_W)
    def _adj(j):
        for k in range(IDX_W // L):
            sl = pl.ds(k * L, L)
            segidx[j, sl] = segidx[j, sl] + my_acc0

    @pl.loop(0, CH // L)
    def _zz(i):
        zrow[pl.ds(i * L, L)] = zero16

    base_row = (c * NS + s) * (CHUNKS * CH // IDX_W)

    @pl.loop(0, CHUNKS)
    def _chunk(g):
        row0 = base_row + g * (CH // IDX_W)
        pltpu.sync_copy(srcb_hbm.at[pl.ds(row0, CH // IDX_W)], idxs)
        pltpu.sync_copy(dstb_hbm.at[pl.ds(row0, CH // IDX_W)], idxd)
        descs = []
        for j in range(CH // IDX_W):
            descs.append(pltpu.async_copy(
                u_hbm.at[idxs.at[j]], rows_u.at[pl.ds(j * IDX_W, IDX_W)], semu))
            descs.append(pltpu.async_copy(
                v_hbm.at[idxd.at[j]], rows_v.at[pl.ds(j * IDX_W, IDX_W)], semv))
        for d in descs:
            d.wait()

        @pl.loop(0, CH)
        def _compute(e):
            z = None
            for k in range(H // L):
                sl = pl.ds(k * L, L)
                m = _silu_vec(rows_u[e, sl] + rows_v[e, sl])
                t = m * w2v[k]
                z = t if z is None else z + t
            zbuf[pl.ds(e * L, L)] = z

        pltpu.sync_copy(zrow, acc_sh.at[pl.ds(my_acc0, CH)])
        for j in range(CH * L // IDX_W):
            pltpu.sync_copy(zbuf.at[pl.ds(j * IDX_W, IDX_W)],
                            acc_sh.at[segidx.at[j]], add=True)

        pltpu.sync_copy(acc_sh.at[pl.ds(my_acc0, CH)],
                        out_hbm.at[pl.ds(row0 * IDX_W, CH)])


_edge_rate = pl.kernel(
    _edge_rate_body,
    out_type=jax.ShapeDtypeStruct((BE,), jnp.float32),
    mesh=_mesh,
    compiler_params=pltpu.CompilerParams(use_tc_tiling_on_sc=False),
    scratch_types=[
        pltpu.VMEM((CH // IDX_W, IDX_W), jnp.int32),
        pltpu.VMEM((CH // IDX_W, IDX_W), jnp.int32),
        pltpu.VMEM((CH, H), jnp.float32),
        pltpu.VMEM((CH, H), jnp.float32),
        pltpu.VMEM((CH * L,), jnp.float32),
        pltpu.VMEM((CH,), jnp.float32),
        pltpu.VMEM((CH * L // IDX_W, IDX_W), jnp.int32),
        pltpu.VMEM((H // L, L), jnp.float32),
        pltpu.VMEM_SHARED((NS * CH,), jnp.float32),
        pltpu.SemaphoreType.DMA,
        pltpu.SemaphoreType.DMA,
    ],
)


def kernel(mu, t, context, edge_index, mp_params, edge_w1, edge_b1, edge_w2, edge_b2):
    t_exp = jnp.broadcast_to(t, (B, N))
    base = jnp.stack([mu, t_exp], axis=-1)
    h = jnp.concatenate([base, context], axis=-1).reshape(BN, -1)

    src = edge_index[0]
    dst = edge_index[1]
    offsets = (jnp.arange(B) * N).astype(src.dtype)
    src_b = (src[None, :] + offsets[:, None]).reshape(-1).astype(jnp.int32)
    dst_b = (dst[None, :] + offsets[:, None]).reshape(-1).astype(jnp.int32)
    srcb2 = src_b.reshape(BE // IDX_W, IDX_W)
    dstb2 = dst_b.reshape(BE // IDX_W, IDX_W)

    for (Wm, bm, Wn, bn) in mp_params:
        in_dim = h.shape[1]
        P, Q = _proj2(h, Wm[:in_dim], bm, Wm[in_dim:])
        agg = _edge_agg(P, Q, srcb2, dstb2)
        h = _update(h, agg, Wn, bn)

    U, V = _proj2(h, edge_w1[:H], edge_b1, edge_w1[H:])
    segidx = (jnp.arange(CH * L, dtype=jnp.int32) // L).reshape(-1, IDX_W)
    logits = _edge_rate(U, V, srcb2, dstb2,
                        edge_w2.reshape(H // L, L), segidx)
    rates = jax.nn.softplus(logits + edge_b2[0]).reshape(B, E)

    rm = jnp.zeros((B, N, N), jnp.float32)
    rm = rm.at[:, src, dst].set(rates)
    diag = jnp.arange(N)
    row_sum = rm.sum(axis=-1)
    rm = rm.at[:, diag, diag].set(-row_sum)
    return rm


# speedup vs baseline: 1.1484x; 1.0006x over previous
import functools

import jax
import jax.numpy as jnp
from jax import lax
from jax.experimental import pallas as pl
from jax.experimental.pallas import tpu as pltpu
from jax.experimental.pallas import tpu_sc as plsc

B, N, E, H = 4, 2048, 65536, 64
BN = B * N
BE = B * E
NC, NS, L = 2, 16, 16
ROWS_PER_CORE = BN // NC
CH = 512
CHUNKS = BE // (NC * NS * CH)
IDX_W = 128

_mesh = plsc.VectorSubcoreMesh(core_axis_name="c", subcore_axis_name="s")


def _silu(x):
    return x / (1.0 + jnp.exp(-x))


def _silu_vec(x):
    return x * (1.0 / (1.0 + jnp.exp(-x)))


def _proj2_body(h_ref, wp_ref, bp_ref, wq_ref, p_ref, q_ref):
    x = h_ref[...]
    p_ref[...] = lax.dot_general(x, wp_ref[...], (((1,), (0,)), ((), ())),
                                 preferred_element_type=jnp.float32) + bp_ref[...]
    q_ref[...] = lax.dot_general(x, wq_ref[...], (((1,), (0,)), ((), ())),
                                 preferred_element_type=jnp.float32)


def _proj2(h, wp, bp, wq):
    return pl.pallas_call(
        _proj2_body,
        out_shape=(jax.ShapeDtypeStruct((BN, H), jnp.float32),
                   jax.ShapeDtypeStruct((BN, H), jnp.float32)),
    )(h, wp, bp, wq)


def _update_body(h_ref, a_ref, w_ref, b_ref, o_ref):
    x = jnp.concatenate([h_ref[...], a_ref[...]], axis=1)
    o = lax.dot_general(x, w_ref[...], (((1,), (0,)), ((), ())),
                        preferred_element_type=jnp.float32)
    o_ref[...] = _silu(o + b_ref[...])


def _update(h, agg, w, b):
    return pl.pallas_call(
        _update_body,
        out_shape=jax.ShapeDtypeStruct((BN, H), jnp.float32),
    )(h, agg, w, b)


def _edge_agg_body(p_hbm, q_hbm, srcb_hbm, dstb_hbm, out_hbm,
                   idxs, idxd, idxw, rows_p, rows_q, agg_sh, semp, semq):
    c = lax.axis_index("c")
    s = lax.axis_index("s")
    zero16 = jnp.zeros((L,), jnp.float32)
    my_sh0 = s * (ROWS_PER_CORE // NS)

    @pl.loop(0, ROWS_PER_CORE // NS)
    def _z(r):
        for k in range(H // L):
            rows_p[r, pl.ds(k * L, L)] = zero16
    pltpu.sync_copy(rows_p.at[pl.ds(0, ROWS_PER_CORE // NS)],
                    agg_sh.at[pl.ds(my_sh0, ROWS_PER_CORE // NS)])
    plsc.subcore_barrier()

    base_row = (c * NS + s) * (CHUNKS * CH // IDX_W)
    row_off = c * ROWS_PER_CORE

    @pl.loop(0, CHUNKS)
    def _chunk(g):
        row0 = base_row + g * (CH // IDX_W)
        pltpu.sync_copy(srcb_hbm.at[pl.ds(row0, CH // IDX_W)], idxs)
        pltpu.sync_copy(dstb_hbm.at[pl.ds(row0, CH // IDX_W)], idxd)
        for j in range(CH // IDX_W):
            for k in range(IDX_W // L):
                sl = pl.ds(k * L, L)
                idxw[j, sl] = idxd[j, sl] - row_off
        descs = []
        for j in range(CH // IDX_W):
            descs.append(pltpu.async_copy(
                p_hbm.at[idxs.at[j]], rows_p.at[pl.ds(j * IDX_W, IDX_W)], semp))
            descs.append(pltpu.async_copy(
                q_hbm.at[idxd.at[j]], rows_q.at[pl.ds(j * IDX_W, IDX_W)], semq))
        for d in descs:
            d.wait()

        @pl.loop(0, CH)
        def _compute(r):
            for k in range(H // L):
                sl = pl.ds(k * L, L)
                a = rows_p[r, sl] + rows_q[r, sl]
                rows_p[r, sl] = _silu_vec(a)

        for j in range(CH // IDX_W):
            pltpu.sync_copy(rows_p.at[pl.ds(j * IDX_W, IDX_W)],
                            agg_sh.at[idxw.at[j]], add=True)

    plsc.subcore_barrier()
    pltpu.sync_copy(agg_sh.at[pl.ds(my_sh0, ROWS_PER_CORE // NS)],
                    out_hbm.at[pl.ds(row_off + my_sh0, ROWS_PER_CORE // NS)])


_edge_agg = pl.kernel(
    _edge_agg_body,
    out_type=jax.ShapeDtypeStruct((BN, H), jnp.float32),
    mesh=_mesh,
    compiler_params=pltpu.CompilerParams(use_tc_tiling_on_sc=False),
    scratch_types=[
        pltpu.VMEM((CH // IDX_W, IDX_W), jnp.int32),
        pltpu.VMEM((CH // IDX_W, IDX_W), jnp.int32),
        pltpu.VMEM((CH // IDX_W, IDX_W), jnp.int32),
        pltpu.VMEM((CH, H), jnp.float32),
        pltpu.VMEM((CH, H), jnp.float32),
        pltpu.VMEM_SHARED((ROWS_PER_CORE, H), jnp.float32),
        pltpu.SemaphoreType.DMA,
        pltpu.SemaphoreType.DMA,
    ],
)


def _edge_rate_body(u_hbm, v_hbm, srcb_hbm, dstb_hbm, w2_hbm, segidx_hbm,
                    out_hbm, idxs, idxd, rows_u, rows_v, zbuf, zrow, segidx,
                    w2v, acc_sh, semu, semv):
    c = lax.axis_index("c")
    s = lax.axis_index("s")
    pltpu.sync_copy(w2_hbm, w2v)
    pltpu.sync_copy(segidx_hbm, segidx)
    zero16 = jnp.zeros((L,), jnp.float32)
    my_acc0 = s * CH
    @pl.loop(0, CH * L // IDX_W)
    def _adj(j):
        for k in range(IDX_W // L):
            sl = pl.ds(k * L, L)
            segidx[j, sl] = segidx[j, sl] + my_acc0

    @pl.loop(0, CH // L)
    def _zz(i):
        zrow[pl.ds(i * L, L)] = zero16

    base_row = (c * NS + s) * (CHUNKS * CH // IDX_W)

    @pl.loop(0, CHUNKS)
    def _chunk(g):
        row0 = base_row + g * (CH // IDX_W)
        pltpu.sync_copy(srcb_hbm.at[pl.ds(row0, CH // IDX_W)], idxs)
        pltpu.sync_copy(dstb_hbm.at[pl.ds(row0, CH // IDX_W)], idxd)
        descs = []
        for j in range(CH // IDX_W):
            descs.append(pltpu.async_copy(
                u_hbm.at[idxs.at[j]], rows_u.at[pl.ds(j * IDX_W, IDX_W)], semu))
            descs.append(pltpu.async_copy(
                v_hbm.at[idxd.at[j]], rows_v.at[pl.ds(j * IDX_W, IDX_W)], semv))
        for d in descs:
            d.wait()

        @pl.loop(0, CH)
        def _compute(e):
            z = None
            for k in range(H // L):
                sl = pl.ds(k * L, L)
                m = _silu_vec(rows_u[e, sl] + rows_v[e, sl])
                t = m * w2v[k]
                z = t if z is None else z + t
            zbuf[pl.ds(e * L, L)] = z

        pltpu.sync_copy(zrow, acc_sh.at[pl.ds(my_acc0, CH)])
        for j in range(CH * L // IDX_W):
            pltpu.sync_copy(zbuf.at[pl.ds(j * IDX_W, IDX_W)],
                            acc_sh.at[segidx.at[j]], add=True)

        pltpu.sync_copy(acc_sh.at[pl.ds(my_acc0, CH)],
                        out_hbm.at[pl.ds(row0 * IDX_W, CH)])


_edge_rate = pl.kernel(
    _edge_rate_body,
    out_type=jax.ShapeDtypeStruct((BE,), jnp.float32),
    mesh=_mesh,
    compiler_params=pltpu.CompilerParams(use_tc_tiling_on_sc=False),
    scratch_types=[
        pltpu.VMEM((CH // IDX_W, IDX_W), jnp.int32),
        pltpu.VMEM((CH // IDX_W, IDX_W), jnp.int32),
        pltpu.VMEM((CH, H), jnp.float32),
        pltpu.VMEM((CH, H), jnp.float32),
        pltpu.VMEM((CH * L,), jnp.float32),
        pltpu.VMEM((CH,), jnp.float32),
        pltpu.VMEM((CH * L // IDX_W, IDX_W), jnp.int32),
        pltpu.VMEM((H // L, L), jnp.float32),
        pltpu.VMEM_SHARED((NS * CH,), jnp.float32),
        pltpu.SemaphoreType.DMA,
        pltpu.SemaphoreType.DMA,
    ],
)


RPS = BN // (NC * NS)
ZR = 16
EPS = BE // (NC * NS)


def _rate_mat_body(rates_hbm, fidx_hbm, segidx_hbm, out_hbm, zbuf, fidx, vals,
                   pbuf, accidx, dval, didx, acc_sh, sem):
    c = lax.axis_index("c")
    s = lax.axis_index("s")
    zero16 = jnp.zeros((L,), jnp.float32)
    lane = lax.broadcasted_iota(jnp.int32, (L,), 0)
    row0 = (c * NS + s) * RPS

    @pl.loop(0, ZR * N // L)
    def _z(i):
        zbuf[pl.ds(i * L, L)] = zero16

    @pl.loop(0, RPS // ZR)
    def _zdma(i):
        pltpu.sync_copy(zbuf, out_hbm.at[pl.ds((row0 + i * ZR) * N, ZR * N)])
    plsc.subcore_barrier()

    ebase = (c * NS + s) * (EPS // IDX_W)
    pltpu.sync_copy(fidx_hbm.at[pl.ds(ebase, EPS // IDX_W)], fidx)
    pltpu.sync_copy(rates_hbm.at[pl.ds(ebase, EPS // IDX_W)], vals)
    for j0 in range(0, EPS // IDX_W, 8):
        descs = [pltpu.async_copy(vals.at[j], out_hbm.at[fidx.at[j]], sem)
                 for j in range(j0, j0 + 8)]
        for d in descs:
            d.wait()
    plsc.subcore_barrier()

    my_acc0 = s * RPS
    @pl.loop(0, RPS // L)
    def _za(i):
        pbuf[pl.ds(i * L, L)] = zero16
    pltpu.sync_copy(pbuf.at[pl.ds(0, RPS)], acc_sh.at[pl.ds(my_acc0, RPS)])

    pltpu.sync_copy(segidx_hbm.at[pl.ds(0, ZR * L // IDX_W)], accidx)

    @pl.loop(0, ZR * L // IDX_W)
    def _ai(j):
        for k in range(IDX_W // L):
            sl = pl.ds(k * L, L)
            accidx[j, sl] = accidx[j, sl] + my_acc0

    @pl.loop(0, RPS // ZR)
    def _rs(i):
        pltpu.sync_copy(out_hbm.at[pl.ds((row0 + i * ZR) * N, ZR * N)], zbuf)

        @pl.loop(0, ZR)
        def _row(r):
            z = None
            for k in range(N // L):
                t = zbuf[pl.ds(r * N + k * L, L)]
                z = t if z is None else z + t
            pbuf[pl.ds(r * L, L)] = z

        for j in range(ZR * L // IDX_W):
            pltpu.sync_copy(pbuf.at[pl.ds(j * IDX_W, IDX_W)],
                            acc_sh.at[accidx.at[j]], add=True)

        @pl.loop(0, ZR * L // IDX_W)
        def _bump(j):
            for k in range(IDX_W // L):
                sl = pl.ds(k * L, L)
                accidx[j, sl] = accidx[j, sl] + ZR

    pltpu.sync_copy(acc_sh.at[pl.ds(my_acc0, RPS)], dval)
    col0 = row0 - (row0 // N) * N

    @pl.loop(0, RPS // L)
    def _dv(q):
        dval[pl.ds(q * L, L)] = -dval[pl.ds(q * L, L)]

    @pl.loop(0, RPS // IDX_W)
    def _di(j):
        for k in range(IDX_W // L):
            base = row0 * N + col0 + (j * IDX_W + k * L) * (N + 1)
            didx[j, pl.ds(k * L, L)] = lane * (N + 1) + base

    for j in range(RPS // IDX_W):
        pltpu.sync_copy(dval.at[pl.ds(j * IDX_W, IDX_W)],
                        out_hbm.at[didx.at[j]])


_rate_mat = pl.kernel(
    _rate_mat_body,
    out_type=jax.ShapeDtypeStruct((BN * N,), jnp.float32),
    mesh=_mesh,
    compiler_params=pltpu.CompilerParams(use_tc_tiling_on_sc=False),
    scratch_types=[
        pltpu.VMEM((ZR * N,), jnp.float32),
        pltpu.VMEM((EPS // IDX_W, IDX_W), jnp.int32),
        pltpu.VMEM((EPS // IDX_W, IDX_W), jnp.float32),
        pltpu.VMEM((RPS,), jnp.float32),
        pltpu.VMEM((ZR * L // IDX_W, IDX_W), jnp.int32),
        pltpu.VMEM((RPS,), jnp.float32),
        pltpu.VMEM((RPS // IDX_W, IDX_W), jnp.int32),
        pltpu.VMEM_SHARED((NS * RPS,), jnp.float32),
        pltpu.SemaphoreType.DMA,
    ],
)


def kernel(mu, t, context, edge_index, mp_params, edge_w1, edge_b1, edge_w2, edge_b2):
    t_exp = jnp.broadcast_to(t, (B, N))
    base = jnp.stack([mu, t_exp], axis=-1)
    h = jnp.concatenate([base, context], axis=-1).reshape(BN, -1)

    src = edge_index[0]
    dst = edge_index[1]
    offsets = (jnp.arange(B) * N).astype(src.dtype)
    src_b = (src[None, :] + offsets[:, None]).reshape(-1).astype(jnp.int32)
    dst_b = (dst[None, :] + offsets[:, None]).reshape(-1).astype(jnp.int32)
    srcb2 = src_b.reshape(BE // IDX_W, IDX_W)
    dstb2 = dst_b.reshape(BE // IDX_W, IDX_W)

    for (Wm, bm, Wn, bn) in mp_params:
        in_dim = h.shape[1]
        P, Q = _proj2(h, Wm[:in_dim], bm, Wm[in_dim:])
        agg = _edge_agg(P, Q, srcb2, dstb2)
        h = _update(h, agg, Wn, bn)

    U, V = _proj2(h, edge_w1[:H], edge_b1, edge_w1[H:])
    segidx = (jnp.arange(CH * L, dtype=jnp.int32) // L).reshape(-1, IDX_W)
    logits = _edge_rate(U, V, srcb2, dstb2,
                        edge_w2.reshape(H // L, L), segidx)
    rates = jax.nn.softplus(logits + edge_b2[0]).reshape(B, E)

    rm = jnp.zeros((B, N, N), jnp.float32).at[:, src, dst].set(rates)
    diag = jnp.arange(N)
    row_sum = rm.sum(axis=-1)
    rm = rm.at[:, diag, diag].set(-row_sum)
    return rm
